# baseline (speedup 1.0000x reference)
import functools

import jax
import jax.numpy as jnp
from jax import lax
from jax.experimental import pallas as pl
from jax.experimental.pallas import tpu as pltpu
from jax.experimental.pallas import tpu_sc as plsc
from jax._src.pallas import mpmd as _mpmd

_BETA = 0.9
_A = 0.2
_P = 1.5
_Q = -50.0

_NC = 2
_NS = 16
_NW = _NC * _NS
_CH = 128


def _loss_body(logits_ref, tgt_ref, loss_ref):
    x = logits_ref[...]
    t = tgt_ref[...]
    m = jnp.max(x, axis=-1)
    lse = jnp.log(jnp.sum(jnp.exp(x - m[:, None]), axis=-1)) + m
    col = lax.broadcasted_iota(jnp.int32, x.shape, 1)
    tl = jnp.sum(jnp.where(col == t[:, None], x, 0.0), axis=-1)
    loss_ref[...] = lse - tl


def _finish_body(loss_ref, g_ref, dpm_ref, scal_ref, newv_ref, out_ref):
    new = g_ref[...] * _BETA + loss_ref[...] * (1.0 - _BETA)
    newv_ref[...] = new
    inv_bias = scal_ref[0]
    gamma = scal_ref[1]
    scaled = new * inv_bias
    k1 = jnp.sum(scaled) * (1.0 / new.shape[0])
    out_ref[...] = (scaled - gamma * k1) / dpm_ref[...]


def _gather_body(b_per_w, exp_hbm, idx_hbm, out_hbm, idx_v, vals_v, sem):
    wid = lax.axis_index("s") * _NC + lax.axis_index("c")
    base = wid * b_per_w
    pltpu.sync_copy(idx_hbm.at[pl.ds(base, b_per_w)], idx_v)
    handles = []
    for j in range(b_per_w // _CH):
        handles.append(pltpu.async_copy(
            exp_hbm.at[idx_v.at[pl.ds(j * _CH, _CH)]],
            vals_v.at[pl.ds(j * _CH, _CH)], sem))
    for h in handles:
        h.wait()
    pltpu.sync_copy(vals_v, out_hbm.at[pl.ds(base, b_per_w)])


def _scatter_body(n_ch, exp_in_hbm, idx_hbm, vals_hbm, out_hbm, idx_v, vals_v,
                  sem):
    del exp_in_hbm
    wid = lax.axis_index("s") * _NC + lax.axis_index("c")
    pltpu.sync_copy(idx_hbm.at[wid], idx_v)
    pltpu.sync_copy(vals_hbm.at[wid], vals_v)
    handles = []
    for j in range(n_ch):
        handles.append(pltpu.async_copy(
            vals_v.at[j], out_hbm.at[idx_v.at[j]], sem))
    for h in handles:
        h.wait()


def kernel(logits, targets, data_parameter_minibatch, exp_avg, index_dataset,
           epoch):
    B, C = logits.shape
    M = exp_avg.shape[0]
    targets = targets.astype(jnp.int32)
    index_dataset = index_dataset.astype(jnp.int32)

    ep = jnp.asarray(epoch, jnp.float32)
    gamma = _A * jnp.tanh(_P * (ep - _Q)) + _A + 1.0
    inv_bias = 1.0 / (1.0 - _BETA ** (ep + 1.0))
    scal = jnp.stack([inv_bias, gamma])

    BM = 512
    grid = B // BM
    loss = pl.pallas_call(
        _loss_body,
        grid=(grid,),
        in_specs=[
            pl.BlockSpec((BM, C), lambda i: (i, 0)),
            pl.BlockSpec((BM,), lambda i: (i,)),
        ],
        out_specs=pl.BlockSpec((BM,), lambda i: (i,)),
        out_shape=jax.ShapeDtypeStruct((B,), jnp.float32),
    )(logits, targets)

    b_per_w = B // _NW
    mesh = plsc.VectorSubcoreMesh(core_axis_name="c", subcore_axis_name="s")
    gathered = pl.kernel(
        functools.partial(_gather_body, b_per_w),
        out_type=jax.ShapeDtypeStruct((B,), jnp.float32),
        mesh=mesh,
        scratch_types=[
            pltpu.VMEM((b_per_w,), jnp.int32),
            pltpu.VMEM((b_per_w,), jnp.float32),
            pltpu.SemaphoreType.DMA,
        ],
    )(exp_avg, index_dataset)

    new_vals, new_loss = pl.pallas_call(
        _finish_body,
        in_specs=[
            pl.BlockSpec(memory_space=pltpu.VMEM),
            pl.BlockSpec(memory_space=pltpu.VMEM),
            pl.BlockSpec(memory_space=pltpu.VMEM),
            pl.BlockSpec(memory_space=pltpu.SMEM),
        ],
        out_specs=[
            pl.BlockSpec(memory_space=pltpu.VMEM),
            pl.BlockSpec(memory_space=pltpu.VMEM),
        ],
        out_shape=[
            jax.ShapeDtypeStruct((B,), jnp.float32),
            jax.ShapeDtypeStruct((B,), jnp.float32),
        ],
    )(loss, gathered, data_parameter_minibatch, scal)

    n_ch = b_per_w // _CH
    idx3 = index_dataset.reshape(_NW, n_ch, _CH)
    vals3 = new_vals.reshape(_NW, n_ch, _CH)
    scatter = _mpmd._mpmd_map(
        [(mesh, functools.partial(_scatter_body, n_ch))],
        jax.ShapeDtypeStruct((M,), jnp.float32),
        input_output_aliases={0: 0},
        scratch_types=[
            pltpu.VMEM((n_ch, _CH), jnp.int32),
            pltpu.VMEM((n_ch, _CH), jnp.float32),
            pltpu.SemaphoreType.DMA,
        ],
    )
    exp_avg_new = scatter(exp_avg, idx3, vals3)

    return new_loss, exp_avg_new


# speedup vs baseline: 2.1102x; 2.1102x over previous
import functools

import jax
import jax.numpy as jnp
from jax import lax
from jax.experimental import pallas as pl
from jax.experimental.pallas import tpu as pltpu
from jax.experimental.pallas import tpu_sc as plsc

_BETA = 0.9
_A = 0.2
_P = 1.5
_Q = -50.0

_NC = 2
_NS = 16
_NW = _NC * _NS
_CH = 128
_L = 16


def _loss_body(logits_ref, tgt_ref, loss_ref):
    x = logits_ref[...]
    t = tgt_ref[...]
    m = jnp.max(x, axis=0)
    lse = jnp.log(jnp.sum(jnp.exp(x - m[None, :]), axis=0)) + m
    row = lax.broadcasted_iota(jnp.int32, x.shape, 0)
    tl = jnp.sum(jnp.where(row == t[None, :], x, 0.0), axis=0)
    loss_ref[...] = lse - tl


def _finish_body(loss_ref, g_ref, dpm_ref, scal_ref, out_ref):
    new = g_ref[...] * _BETA + loss_ref[...] * (1.0 - _BETA)
    inv_bias = scal_ref[0]
    gamma = scal_ref[1]
    scaled = new * inv_bias
    k1 = jnp.sum(scaled) * (1.0 / new.shape[0])
    out_ref[...] = (scaled - gamma * k1) / dpm_ref[...]


def _gather_body(b_per_w, exp_hbm, idx_hbm, out_hbm, idx_v, vals_v, sem):
    wid = lax.axis_index("s") * _NC + lax.axis_index("c")
    base = wid * b_per_w
    pltpu.sync_copy(idx_hbm.at[pl.ds(base, b_per_w)], idx_v)
    handles = []
    for j in range(b_per_w // _CH):
        handles.append(pltpu.async_copy(
            exp_hbm.at[idx_v.at[pl.ds(j * _CH, _CH)]],
            vals_v.at[pl.ds(j * _CH, _CH)], sem))
    for h in handles:
        h.wait()
    pltpu.sync_copy(vals_v, out_hbm.at[pl.ds(base, b_per_w)])


def _scatter_body(M, B, exp_hbm, idx_hbm, loss_hbm, gath_hbm, out_hbm,
                  idx_v, loss_v, gath_v, idx2, vals2, tbuf, buf_sh, sem):
    half = M // 2
    cid = lax.axis_index("c")
    sid = lax.axis_index("s")
    base = cid * half
    seg = (half // _NS) // 8 * 8
    seg15 = half - (_NS - 1) * seg
    off = sid * seg

    @pl.when(sid < _NS - 1)
    def _():
        pltpu.sync_copy(exp_hbm.at[pl.ds(base + off, seg)],
                        tbuf.at[pl.ds(0, seg)])
        pltpu.sync_copy(tbuf.at[pl.ds(0, seg)], buf_sh.at[pl.ds(off, seg)])

    @pl.when(sid == _NS - 1)
    def _():
        pltpu.sync_copy(exp_hbm.at[pl.ds(base + off, seg15)],
                        tbuf.at[pl.ds(0, seg15)])
        pltpu.sync_copy(tbuf.at[pl.ds(0, seg15)], buf_sh.at[pl.ds(off, seg15)])

    per_tile = B // _NS
    pbase = sid * per_tile
    pltpu.sync_copy(idx_hbm.at[pl.ds(pbase, per_tile)], idx_v)
    pltpu.sync_copy(loss_hbm.at[pl.ds(pbase, per_tile)], loss_v)
    pltpu.sync_copy(gath_hbm.at[pl.ds(pbase, per_tile)], gath_v)

    base_v = base
    for k in range(per_tile // _L):
        r, c0 = k // (_CH // _L), (k % (_CH // _L)) * _L
        idx = idx_v[pl.ds(k * _L, _L)]
        new = (gath_v[pl.ds(k * _L, _L)] * _BETA
               + loss_v[pl.ds(k * _L, _L)] * (1.0 - _BETA))
        local = idx - base_v
        ok = (local >= 0) & (local < half)
        idx2[r, pl.ds(c0, _L)] = jnp.where(ok, local, half)
        vals2[r, pl.ds(c0, _L)] = new

    plsc.subcore_barrier()

    handles = []
    for j in range(per_tile // _CH):
        handles.append(pltpu.async_copy(
            vals2.at[j], buf_sh.at[idx2.at[j]], sem))
    for h in handles:
        h.wait()

    plsc.subcore_barrier()

    @pl.when(sid < _NS - 1)
    def _():
        pltpu.sync_copy(buf_sh.at[pl.ds(off, seg)], tbuf.at[pl.ds(0, seg)])
        pltpu.sync_copy(tbuf.at[pl.ds(0, seg)],
                        out_hbm.at[pl.ds(base + off, seg)])

    @pl.when(sid == _NS - 1)
    def _():
        pltpu.sync_copy(buf_sh.at[pl.ds(off, seg15)], tbuf.at[pl.ds(0, seg15)])
        pltpu.sync_copy(tbuf.at[pl.ds(0, seg15)],
                        out_hbm.at[pl.ds(base + off, seg15)])


def _scatter_body_wrap(M, B, *refs):
    return _scatter_body(M, B, *refs)


def kernel(logits, targets, data_parameter_minibatch, exp_avg, index_dataset,
           epoch):
    B, C = logits.shape
    M = exp_avg.shape[0]
    targets = targets.astype(jnp.int32)
    index_dataset = index_dataset.astype(jnp.int32)

    ep = jnp.asarray(epoch, jnp.float32)
    gamma = _A * jnp.tanh(_P * (ep - _Q)) + _A + 1.0
    inv_bias = 1.0 / (1.0 - _BETA ** (ep + 1.0))
    scal = jnp.stack([inv_bias, gamma])

    logits_t = jnp.swapaxes(logits, 0, 1)
    BN = 512
    grid = B // BN
    loss = pl.pallas_call(
        _loss_body,
        grid=(grid,),
        in_specs=[
            pl.BlockSpec((C, BN), lambda i: (0, i)),
            pl.BlockSpec((BN,), lambda i: (i,)),
        ],
        out_specs=pl.BlockSpec((BN,), lambda i: (i,)),
        out_shape=jax.ShapeDtypeStruct((B,), jnp.float32),
    )(logits_t, targets)

    b_per_w = B // _NW
    mesh = plsc.VectorSubcoreMesh(core_axis_name="c", subcore_axis_name="s")
    gathered = pl.kernel(
        functools.partial(_gather_body, b_per_w),
        out_type=jax.ShapeDtypeStruct((B,), jnp.float32),
        mesh=mesh,
        scratch_types=[
            pltpu.VMEM((b_per_w,), jnp.int32),
            pltpu.VMEM((b_per_w,), jnp.float32),
            pltpu.SemaphoreType.DMA,
        ],
    )(exp_avg, index_dataset)

    new_loss = pl.pallas_call(
        _finish_body,
        in_specs=[
            pl.BlockSpec(memory_space=pltpu.VMEM),
            pl.BlockSpec(memory_space=pltpu.VMEM),
            pl.BlockSpec(memory_space=pltpu.VMEM),
            pl.BlockSpec(memory_space=pltpu.SMEM),
        ],
        out_specs=pl.BlockSpec(memory_space=pltpu.VMEM),
        out_shape=jax.ShapeDtypeStruct((B,), jnp.float32),
    )(loss, gathered, data_parameter_minibatch, scal)

    per_tile = B // _NS
    exp_avg_new = pl.kernel(
        functools.partial(_scatter_body_wrap, M, B),
        out_type=jax.ShapeDtypeStruct((M,), jnp.float32),
        mesh=mesh,
        scratch_types=[
            pltpu.VMEM((per_tile,), jnp.int32),
            pltpu.VMEM((per_tile,), jnp.float32),
            pltpu.VMEM((per_tile,), jnp.float32),
            pltpu.VMEM((per_tile // _CH, _CH), jnp.int32),
            pltpu.VMEM((per_tile // _CH, _CH), jnp.float32),
            pltpu.VMEM((M // 2 - (_NS - 1) * ((M // 2 // _NS) // 8 * 8),),
                       jnp.float32),
            pltpu.VMEM_SHARED((M // 2 + 8,), jnp.float32),
            pltpu.SemaphoreType.DMA,
        ],
    )(exp_avg, index_dataset, loss, gathered)

    return new_loss, exp_avg_new
